# baseline (speedup 1.0000x reference)
import functools

import jax
import jax.numpy as jnp
from jax import lax
from jax.experimental import pallas as pl
from jax.experimental.pallas import tpu as pltpu
from jax.experimental.pallas import tpu_sc as plsc

EMBED = 64
BATCH = 16384
NROWS = 1000000
NC = 2
NS = 16
NW = NC * NS
B_PER_W = BATCH // NW
IDX_CHUNK = 128
N_CHUNKS = B_PER_W // IDX_CHUNK

PACK_L = 2048


def _pack_body(u_ref, i_ref, out_ref):
    x = jnp.concatenate([u_ref[...], i_ref[...]], axis=0)
    xt = jnp.swapaxes(x, 0, 1)
    xt3 = xt.reshape(PACK_L // 2, 2, 2 * EMBED)
    hi = xt3[:, 0, :].astype(jnp.bfloat16)
    lo = xt3[:, 1, :].astype(jnp.bfloat16)
    hib = lax.bitcast_convert_type(hi, jnp.uint16).astype(jnp.uint32)
    lob = lax.bitcast_convert_type(lo, jnp.uint16).astype(jnp.uint32)
    out_ref[...] = (hib << 16) | lob


def _pack(u_t, i_t):
    grid = (pl.cdiv(NROWS, PACK_L),)
    return pl.pallas_call(
        _pack_body,
        grid=grid,
        in_specs=[
            pl.BlockSpec((EMBED, PACK_L), lambda k: (0, k)),
            pl.BlockSpec((EMBED, PACK_L), lambda k: (0, k)),
        ],
        out_specs=pl.BlockSpec((PACK_L // 2, 2 * EMBED), lambda k: (k, 0)),
        out_shape=jax.ShapeDtypeStruct((NROWS // 2, 2 * EMBED), jnp.uint32),
    )(u_t, i_t)


def _gather_body(uid_hbm, iid_hbm, packed_hbm, xu_hbm, xi_hbm,
                 uidx_v, iidx_v, rows_v, sem):
    wid = lax.axis_index("s") * NC + lax.axis_index("c")
    base = wid * B_PER_W
    row_base = wid * N_CHUNKS
    pltpu.sync_copy(uid_hbm.at[pl.ds(row_base, N_CHUNKS)], uidx_v)
    pltpu.sync_copy(iid_hbm.at[pl.ds(row_base, N_CHUNKS)], iidx_v)
    for idx_v, dst in ((uidx_v, xu_hbm), (iidx_v, xi_hbm)):
        copies = []
        for j in range(N_CHUNKS):
            copies.append(pltpu.async_copy(
                packed_hbm.at[idx_v.at[j]],
                rows_v.at[pl.ds(j * IDX_CHUNK, IDX_CHUNK)], sem))
        for c in copies:
            c.wait()
        pltpu.sync_copy(rows_v, dst.at[pl.ds(base, B_PER_W)])


@functools.cache
def _make_gather():
    return pl.kernel(
        _gather_body,
        mesh=plsc.VectorSubcoreMesh(core_axis_name="c", subcore_axis_name="s"),
        out_type=[
            jax.ShapeDtypeStruct((BATCH, 2 * EMBED), jnp.uint32),
            jax.ShapeDtypeStruct((BATCH, 2 * EMBED), jnp.uint32),
        ],
        scratch_types=[
            pltpu.VMEM((N_CHUNKS, IDX_CHUNK), jnp.int32),
            pltpu.VMEM((N_CHUNKS, IDX_CHUNK), jnp.int32),
            pltpu.VMEM((B_PER_W, 2 * EMBED), jnp.uint32),
            pltpu.SemaphoreType.DMA,
        ],
    )


def _select_rows(xraw, parity):
    sel = jnp.where(parity == 1, xraw << 16, xraw)
    sel = sel & jnp.uint32(0xFFFF0000)
    return lax.bitcast_convert_type(sel, jnp.float32)


def _mlp_body(xu_ref, xi_ref, hu_ref, hi_ref, wa_ref, wb_ref, b1_ref,
              w2_ref, b2_ref, w3_ref, b3_ref, out_ref):
    xu = _select_rows(xu_ref[...], hu_ref[...])
    xi = _select_rows(xi_ref[...], hi_ref[...])
    h = jnp.dot(xu, wa_ref[...], preferred_element_type=jnp.float32)
    h = h + jnp.dot(xi, wb_ref[...], preferred_element_type=jnp.float32)
    h = jnp.maximum(h + b1_ref[...], 0.0)
    h = jnp.maximum(
        jnp.dot(h, w2_ref[...], preferred_element_type=jnp.float32)
        + b2_ref[...], 0.0)
    out_ref[...] = jnp.sum(h * w3_ref[...], axis=1) + b3_ref[0]


MLP_BLK = 2048


def _mlp(xu, xi, hu, hi, wa, wb, b1, w2t, b2, w3, b3):
    grid = (BATCH // MLP_BLK,)
    return pl.pallas_call(
        _mlp_body,
        grid=grid,
        in_specs=[
            pl.BlockSpec((MLP_BLK, 2 * EMBED), lambda i: (i, 0)),
            pl.BlockSpec((MLP_BLK, 2 * EMBED), lambda i: (i, 0)),
            pl.BlockSpec((MLP_BLK, 1), lambda i: (i, 0)),
            pl.BlockSpec((MLP_BLK, 1), lambda i: (i, 0)),
            pl.BlockSpec((2 * EMBED, 128), lambda i: (0, 0)),
            pl.BlockSpec((2 * EMBED, 128), lambda i: (0, 0)),
            pl.BlockSpec((1, 128), lambda i: (0, 0)),
            pl.BlockSpec((128, EMBED), lambda i: (0, 0)),
            pl.BlockSpec((1, EMBED), lambda i: (0, 0)),
            pl.BlockSpec((1, EMBED), lambda i: (0, 0)),
            pl.BlockSpec(memory_space=pltpu.SMEM),
        ],
        out_specs=pl.BlockSpec((MLP_BLK,), lambda i: (i,)),
        out_shape=jax.ShapeDtypeStruct((BATCH,), jnp.float32),
    )(xu, xi, hu, hi, wa, wb, b1, w2t, b2, w3, b3)


def kernel(user_ids, item_ids, user_table, item_table, W1, b1, W2, b2, W3, b3):
    packed = _pack(user_table.T, item_table.T)
    uid = user_ids.astype(jnp.int32)
    iid = item_ids.astype(jnp.int32)
    uq2 = (uid >> 1).reshape(BATCH // IDX_CHUNK, IDX_CHUNK)
    iq2 = (iid >> 1).reshape(BATCH // IDX_CHUNK, IDX_CHUNK)
    xu, xi = _make_gather()(uq2, iq2, packed)
    hu = (uid & 1).astype(jnp.uint32).reshape(BATCH, 1)
    hi = (iid & 1).astype(jnp.uint32).reshape(BATCH, 1)
    zeros = jnp.zeros((EMBED, 128), jnp.float32)
    wa = jnp.concatenate([W1[:, :EMBED].T, zeros], axis=0)
    wb = jnp.concatenate([zeros, W1[:, EMBED:].T], axis=0)
    return _mlp(xu, xi, hu, hi, wa, wb, b1.reshape(1, 128), W2.T,
                b2.reshape(1, EMBED), W3, b3)


# speedup vs baseline: 1.4264x; 1.4264x over previous
import functools

import jax
import jax.numpy as jnp
from jax import lax
from jax.experimental import pallas as pl
from jax.experimental.pallas import tpu as pltpu
from jax.experimental.pallas import tpu_sc as plsc

EMBED = 64
BATCH = 16384
NROWS = 1000000
NC = 2
NS = 16
NW = NC * NS
B_PER_W = BATCH // NW
IDX_CHUNK = 128
N_CHUNKS = B_PER_W // IDX_CHUNK

PACK_L = 2048
PACK_GRID = -(-NROWS // PACK_L)
NPAIR = PACK_GRID * (PACK_L // 2)


def _pack_body(u_ref, i_ref, out_ref):
    x = jnp.concatenate([u_ref[...], i_ref[...]], axis=0)
    xt = jnp.swapaxes(x, 0, 1)
    w = lax.bitcast_convert_type(xt, jnp.uint32)
    t = w + jnp.uint32(0x7FFF) + ((w >> 16) & jnp.uint32(1))
    hi = t[: PACK_L // 2, :]
    lo = t[PACK_L // 2:, :]
    out_ref[...] = (hi & jnp.uint32(0xFFFF0000)) | (lo >> 16)


def _pack(u_t, i_t):
    return pl.pallas_call(
        _pack_body,
        grid=(PACK_GRID,),
        in_specs=[
            pl.BlockSpec((EMBED, PACK_L), lambda k: (0, k)),
            pl.BlockSpec((EMBED, PACK_L), lambda k: (0, k)),
        ],
        out_specs=pl.BlockSpec((PACK_L // 2, 2 * EMBED), lambda k: (k, 0)),
        out_shape=jax.ShapeDtypeStruct((NPAIR, 2 * EMBED), jnp.uint32),
    )(u_t, i_t)


def _gather_body(uid_hbm, iid_hbm, packed_hbm, xu_hbm, xi_hbm,
                 uidx_v, iidx_v, rows_v, sem):
    wid = lax.axis_index("s") * NC + lax.axis_index("c")
    base = wid * B_PER_W
    row_base = wid * N_CHUNKS
    pltpu.sync_copy(uid_hbm.at[pl.ds(row_base, N_CHUNKS)], uidx_v)
    pltpu.sync_copy(iid_hbm.at[pl.ds(row_base, N_CHUNKS)], iidx_v)
    for idx_v, dst in ((uidx_v, xu_hbm), (iidx_v, xi_hbm)):
        copies = []
        for j in range(N_CHUNKS):
            copies.append(pltpu.async_copy(
                packed_hbm.at[idx_v.at[j]],
                rows_v.at[pl.ds(j * IDX_CHUNK, IDX_CHUNK)], sem))
        for c in copies:
            c.wait()
        pltpu.sync_copy(rows_v, dst.at[pl.ds(base, B_PER_W)])


@functools.cache
def _make_gather():
    return pl.kernel(
        _gather_body,
        mesh=plsc.VectorSubcoreMesh(core_axis_name="c", subcore_axis_name="s"),
        out_type=[
            jax.ShapeDtypeStruct((BATCH, 2 * EMBED), jnp.uint32),
            jax.ShapeDtypeStruct((BATCH, 2 * EMBED), jnp.uint32),
        ],
        scratch_types=[
            pltpu.VMEM((N_CHUNKS, IDX_CHUNK), jnp.int32),
            pltpu.VMEM((N_CHUNKS, IDX_CHUNK), jnp.int32),
            pltpu.VMEM((B_PER_W, 2 * EMBED), jnp.uint32),
            pltpu.SemaphoreType.DMA,
        ],
    )


def _select_rows(xraw, parity):
    sel = jnp.where(parity == 1, xraw << 16, xraw)
    sel = sel & jnp.uint32(0xFFFF0000)
    return lax.bitcast_convert_type(sel, jnp.float32)


def _mlp_body(xu_ref, xi_ref, hu_ref, hi_ref, wa_ref, wb_ref, b1_ref,
              w2_ref, b2_ref, w3_ref, b3_ref, out_ref):
    xu = _select_rows(xu_ref[...], hu_ref[...])
    xi = _select_rows(xi_ref[...], hi_ref[...])
    h = jnp.dot(xu, wa_ref[...], preferred_element_type=jnp.float32)
    h = h + jnp.dot(xi, wb_ref[...], preferred_element_type=jnp.float32)
    h = jnp.maximum(h + b1_ref[...], 0.0)
    h = jnp.maximum(
        jnp.dot(h, w2_ref[...], preferred_element_type=jnp.float32)
        + b2_ref[...], 0.0)
    out_ref[...] = jnp.sum(h * w3_ref[...], axis=1) + b3_ref[0]


MLP_BLK = 2048


def _mlp(xu, xi, hu, hi, wa, wb, b1, w2t, b2, w3, b3):
    grid = (BATCH // MLP_BLK,)
    return pl.pallas_call(
        _mlp_body,
        grid=grid,
        in_specs=[
            pl.BlockSpec((MLP_BLK, 2 * EMBED), lambda i: (i, 0)),
            pl.BlockSpec((MLP_BLK, 2 * EMBED), lambda i: (i, 0)),
            pl.BlockSpec((MLP_BLK, 1), lambda i: (i, 0)),
            pl.BlockSpec((MLP_BLK, 1), lambda i: (i, 0)),
            pl.BlockSpec((2 * EMBED, 128), lambda i: (0, 0)),
            pl.BlockSpec((2 * EMBED, 128), lambda i: (0, 0)),
            pl.BlockSpec((1, 128), lambda i: (0, 0)),
            pl.BlockSpec((128, EMBED), lambda i: (0, 0)),
            pl.BlockSpec((1, EMBED), lambda i: (0, 0)),
            pl.BlockSpec((1, EMBED), lambda i: (0, 0)),
            pl.BlockSpec(memory_space=pltpu.SMEM),
        ],
        out_specs=pl.BlockSpec((MLP_BLK,), lambda i: (i,)),
        out_shape=jax.ShapeDtypeStruct((BATCH,), jnp.float32),
    )(xu, xi, hu, hi, wa, wb, b1, w2t, b2, w3, b3)


def kernel(user_ids, item_ids, user_table, item_table, W1, b1, W2, b2, W3, b3):
    packed = _pack(user_table.T, item_table.T)
    uid = user_ids.astype(jnp.int32)
    iid = item_ids.astype(jnp.int32)
    uq = ((uid >> 11) << 10) | (uid & 1023)
    iq = ((iid >> 11) << 10) | (iid & 1023)
    uq2 = uq.reshape(BATCH // IDX_CHUNK, IDX_CHUNK)
    iq2 = iq.reshape(BATCH // IDX_CHUNK, IDX_CHUNK)
    xu, xi = _make_gather()(uq2, iq2, packed)
    hu = ((uid >> 10) & 1).astype(jnp.uint32).reshape(BATCH, 1)
    hi = ((iid >> 10) & 1).astype(jnp.uint32).reshape(BATCH, 1)
    zeros = jnp.zeros((EMBED, 128), jnp.float32)
    wa = jnp.concatenate([W1[:, :EMBED].T, zeros], axis=0)
    wb = jnp.concatenate([zeros, W1[:, EMBED:].T], axis=0)
    return _mlp(xu, xi, hu, hi, wa, wb, b1.reshape(1, 128), W2.T,
                b2.reshape(1, EMBED), W3, b3)


# speedup vs baseline: 2.3004x; 1.6127x over previous
import functools

import jax
import jax.numpy as jnp
from jax import lax
from jax.experimental import pallas as pl
from jax.experimental.pallas import tpu as pltpu
from jax.experimental.pallas import tpu_sc as plsc

EMBED = 64
BATCH = 16384
NROWS = 1000000
NC = 2
NS = 16
NW = NC * NS
B_PER_W = BATCH // NW
IDX_CHUNK = 128
N_CHUNKS = B_PER_W // IDX_CHUNK

PACK_L = 8192
PACK_GRID = -(-NROWS // PACK_L)
NPAIR = PACK_GRID * (PACK_L // 2)
PACK_HALF = PACK_L // 2
SH_HALF = PACK_HALF.bit_length() - 1
SH_BLK = SH_HALF + 1


def _pack_body(u_ref, i_ref, out_ref):
    x = jnp.concatenate([u_ref[...], i_ref[...]], axis=0)
    xt = jnp.swapaxes(x, 0, 1)
    w = lax.bitcast_convert_type(xt, jnp.uint32)
    t = w + jnp.uint32(0x7FFF) + ((w >> 16) & jnp.uint32(1))
    hi = t[: PACK_L // 2, :]
    lo = t[PACK_L // 2:, :]
    out_ref[...] = (hi & jnp.uint32(0xFFFF0000)) | (lo >> 16)


def _pack(u_t, i_t):
    return pl.pallas_call(
        _pack_body,
        grid=(PACK_GRID,),
        in_specs=[
            pl.BlockSpec((EMBED, PACK_L), lambda k: (0, k)),
            pl.BlockSpec((EMBED, PACK_L), lambda k: (0, k)),
        ],
        out_specs=pl.BlockSpec((PACK_L // 2, 2 * EMBED), lambda k: (k, 0)),
        out_shape=jax.ShapeDtypeStruct((NPAIR, 2 * EMBED), jnp.uint32),
    )(u_t, i_t)


def _gather_body(uid_hbm, iid_hbm, packed_hbm, xu_hbm, xi_hbm,
                 uidx_v, iidx_v, rows_v, sem):
    wid = lax.axis_index("s") * NC + lax.axis_index("c")
    base = wid * B_PER_W
    row_base = wid * N_CHUNKS
    pltpu.sync_copy(uid_hbm.at[pl.ds(row_base, N_CHUNKS)], uidx_v)
    pltpu.sync_copy(iid_hbm.at[pl.ds(row_base, N_CHUNKS)], iidx_v)
    for idx_v, dst in ((uidx_v, xu_hbm), (iidx_v, xi_hbm)):
        copies = []
        for j in range(N_CHUNKS):
            copies.append(pltpu.async_copy(
                packed_hbm.at[idx_v.at[j]],
                rows_v.at[pl.ds(j * IDX_CHUNK, IDX_CHUNK)], sem))
        for c in copies:
            c.wait()
        pltpu.sync_copy(rows_v, dst.at[pl.ds(base, B_PER_W)])


@functools.cache
def _make_gather():
    return pl.kernel(
        _gather_body,
        mesh=plsc.VectorSubcoreMesh(core_axis_name="c", subcore_axis_name="s"),
        out_type=[
            jax.ShapeDtypeStruct((BATCH, 2 * EMBED), jnp.uint32),
            jax.ShapeDtypeStruct((BATCH, 2 * EMBED), jnp.uint32),
        ],
        scratch_types=[
            pltpu.VMEM((N_CHUNKS, IDX_CHUNK), jnp.int32),
            pltpu.VMEM((N_CHUNKS, IDX_CHUNK), jnp.int32),
            pltpu.VMEM((B_PER_W, 2 * EMBED), jnp.uint32),
            pltpu.SemaphoreType.DMA,
        ],
    )


def _select_rows(xraw, parity):
    sel = jnp.where(parity == 1, xraw << 16, xraw)
    sel = sel & jnp.uint32(0xFFFF0000)
    return lax.bitcast_convert_type(sel, jnp.float32)


def _mlp_body(xu_ref, xi_ref, hu_ref, hi_ref, wa_ref, wb_ref, b1_ref,
              w2_ref, b2_ref, w3_ref, b3_ref, out_ref):
    xu = _select_rows(xu_ref[...], hu_ref[...])
    xi = _select_rows(xi_ref[...], hi_ref[...])
    h = jnp.dot(xu, wa_ref[...], preferred_element_type=jnp.float32)
    h = h + jnp.dot(xi, wb_ref[...], preferred_element_type=jnp.float32)
    h = jnp.maximum(h + b1_ref[...], 0.0)
    h = jnp.maximum(
        jnp.dot(h, w2_ref[...], preferred_element_type=jnp.float32)
        + b2_ref[...], 0.0)
    out_ref[...] = jnp.sum(h * w3_ref[...], axis=1) + b3_ref[0]


MLP_BLK = 2048


def _mlp(xu, xi, hu, hi, wa, wb, b1, w2t, b2, w3, b3):
    grid = (BATCH // MLP_BLK,)
    return pl.pallas_call(
        _mlp_body,
        grid=grid,
        in_specs=[
            pl.BlockSpec((MLP_BLK, 2 * EMBED), lambda i: (i, 0)),
            pl.BlockSpec((MLP_BLK, 2 * EMBED), lambda i: (i, 0)),
            pl.BlockSpec((MLP_BLK, 1), lambda i: (i, 0)),
            pl.BlockSpec((MLP_BLK, 1), lambda i: (i, 0)),
            pl.BlockSpec((2 * EMBED, 128), lambda i: (0, 0)),
            pl.BlockSpec((2 * EMBED, 128), lambda i: (0, 0)),
            pl.BlockSpec((1, 128), lambda i: (0, 0)),
            pl.BlockSpec((128, EMBED), lambda i: (0, 0)),
            pl.BlockSpec((1, EMBED), lambda i: (0, 0)),
            pl.BlockSpec((1, EMBED), lambda i: (0, 0)),
            pl.BlockSpec(memory_space=pltpu.SMEM),
        ],
        out_specs=pl.BlockSpec((MLP_BLK,), lambda i: (i,)),
        out_shape=jax.ShapeDtypeStruct((BATCH,), jnp.float32),
    )(xu, xi, hu, hi, wa, wb, b1, w2t, b2, w3, b3)


def kernel(user_ids, item_ids, user_table, item_table, W1, b1, W2, b2, W3, b3):
    packed = _pack(user_table.T, item_table.T)
    uid = user_ids.astype(jnp.int32)
    iid = item_ids.astype(jnp.int32)
    uq = ((uid >> SH_BLK) << SH_HALF) | (uid & (PACK_HALF - 1))
    iq = ((iid >> SH_BLK) << SH_HALF) | (iid & (PACK_HALF - 1))
    uq2 = uq.reshape(BATCH // IDX_CHUNK, IDX_CHUNK)
    iq2 = iq.reshape(BATCH // IDX_CHUNK, IDX_CHUNK)
    xu, xi = _make_gather()(uq2, iq2, packed)
    hu = ((uid >> SH_HALF) & 1).astype(jnp.uint32).reshape(BATCH, 1)
    hi = ((iid >> SH_HALF) & 1).astype(jnp.uint32).reshape(BATCH, 1)
    zeros = jnp.zeros((EMBED, 128), jnp.float32)
    wa = jnp.concatenate([W1[:, :EMBED].T, zeros], axis=0)
    wb = jnp.concatenate([zeros, W1[:, EMBED:].T], axis=0)
    return _mlp(xu, xi, hu, hi, wa, wb, b1.reshape(1, 128), W2.T,
                b2.reshape(1, EMBED), W3, b3)


# speedup vs baseline: 2.4131x; 1.0490x over previous
import functools

import jax
import jax.numpy as jnp
from jax import lax
from jax.experimental import pallas as pl
from jax.experimental.pallas import tpu as pltpu
from jax.experimental.pallas import tpu_sc as plsc

EMBED = 64
BATCH = 16384
NROWS = 1000000
NC = 2
NS = 16
NW = NC * NS
B_PER_W = BATCH // NW
IDX_CHUNK = 128
N_CHUNKS = B_PER_W // IDX_CHUNK

PACK_L = 16384
PACK_GRID = -(-NROWS // PACK_L)
NPAIR = PACK_GRID * (PACK_L // 2)
PACK_HALF = PACK_L // 2
SH_HALF = PACK_HALF.bit_length() - 1
SH_BLK = SH_HALF + 1


def _pack_body(u_ref, i_ref, out_ref):
    x = jnp.concatenate([u_ref[...], i_ref[...]], axis=0)
    xt = jnp.swapaxes(x, 0, 1)
    w = lax.bitcast_convert_type(xt, jnp.uint32)
    t = w + jnp.uint32(0x7FFF) + ((w >> 16) & jnp.uint32(1))
    hi = t[: PACK_L // 2, :]
    lo = t[PACK_L // 2:, :]
    out_ref[...] = (hi & jnp.uint32(0xFFFF0000)) | (lo >> 16)


def _pack(u_t, i_t):
    return pl.pallas_call(
        _pack_body,
        grid=(PACK_GRID,),
        in_specs=[
            pl.BlockSpec((EMBED, PACK_L), lambda k: (0, k)),
            pl.BlockSpec((EMBED, PACK_L), lambda k: (0, k)),
        ],
        out_specs=pl.BlockSpec((PACK_L // 2, 2 * EMBED), lambda k: (k, 0)),
        out_shape=jax.ShapeDtypeStruct((NPAIR, 2 * EMBED), jnp.uint32),
    )(u_t, i_t)


def _gather_body(uid_hbm, iid_hbm, packed_hbm, xu_hbm, xi_hbm,
                 uidx_v, iidx_v, rows_v, sem):
    wid = lax.axis_index("s") * NC + lax.axis_index("c")
    base = wid * B_PER_W
    row_base = wid * N_CHUNKS
    pltpu.sync_copy(uid_hbm.at[pl.ds(row_base, N_CHUNKS)], uidx_v)
    pltpu.sync_copy(iid_hbm.at[pl.ds(row_base, N_CHUNKS)], iidx_v)
    for idx_v, dst in ((uidx_v, xu_hbm), (iidx_v, xi_hbm)):
        copies = []
        for j in range(N_CHUNKS):
            copies.append(pltpu.async_copy(
                packed_hbm.at[idx_v.at[j]],
                rows_v.at[pl.ds(j * IDX_CHUNK, IDX_CHUNK)], sem))
        for c in copies:
            c.wait()
        pltpu.sync_copy(rows_v, dst.at[pl.ds(base, B_PER_W)])


@functools.cache
def _make_gather():
    return pl.kernel(
        _gather_body,
        mesh=plsc.VectorSubcoreMesh(core_axis_name="c", subcore_axis_name="s"),
        out_type=[
            jax.ShapeDtypeStruct((BATCH, 2 * EMBED), jnp.uint32),
            jax.ShapeDtypeStruct((BATCH, 2 * EMBED), jnp.uint32),
        ],
        scratch_types=[
            pltpu.VMEM((N_CHUNKS, IDX_CHUNK), jnp.int32),
            pltpu.VMEM((N_CHUNKS, IDX_CHUNK), jnp.int32),
            pltpu.VMEM((B_PER_W, 2 * EMBED), jnp.uint32),
            pltpu.SemaphoreType.DMA,
        ],
    )


def _select_rows(xraw, parity):
    sel = jnp.where(parity == 1, xraw << 16, xraw)
    sel = sel & jnp.uint32(0xFFFF0000)
    return lax.bitcast_convert_type(sel, jnp.float32)


def _mlp_body(xu_ref, xi_ref, hu_ref, hi_ref, wa_ref, wb_ref, b1_ref,
              w2_ref, b2_ref, w3_ref, b3_ref, out_ref):
    xu = _select_rows(xu_ref[...], hu_ref[...])
    xi = _select_rows(xi_ref[...], hi_ref[...])
    h = jnp.dot(xu, wa_ref[...], preferred_element_type=jnp.float32)
    h = h + jnp.dot(xi, wb_ref[...], preferred_element_type=jnp.float32)
    h = jnp.maximum(h + b1_ref[...], 0.0)
    h = jnp.maximum(
        jnp.dot(h, w2_ref[...], preferred_element_type=jnp.float32)
        + b2_ref[...], 0.0)
    out_ref[...] = jnp.sum(h * w3_ref[...], axis=1) + b3_ref[0]


MLP_BLK = 2048


def _mlp(xu, xi, hu, hi, wa, wb, b1, w2t, b2, w3, b3):
    grid = (BATCH // MLP_BLK,)
    return pl.pallas_call(
        _mlp_body,
        grid=grid,
        in_specs=[
            pl.BlockSpec((MLP_BLK, 2 * EMBED), lambda i: (i, 0)),
            pl.BlockSpec((MLP_BLK, 2 * EMBED), lambda i: (i, 0)),
            pl.BlockSpec((MLP_BLK, 1), lambda i: (i, 0)),
            pl.BlockSpec((MLP_BLK, 1), lambda i: (i, 0)),
            pl.BlockSpec((2 * EMBED, 128), lambda i: (0, 0)),
            pl.BlockSpec((2 * EMBED, 128), lambda i: (0, 0)),
            pl.BlockSpec((1, 128), lambda i: (0, 0)),
            pl.BlockSpec((128, EMBED), lambda i: (0, 0)),
            pl.BlockSpec((1, EMBED), lambda i: (0, 0)),
            pl.BlockSpec((1, EMBED), lambda i: (0, 0)),
            pl.BlockSpec(memory_space=pltpu.SMEM),
        ],
        out_specs=pl.BlockSpec((MLP_BLK,), lambda i: (i,)),
        out_shape=jax.ShapeDtypeStruct((BATCH,), jnp.float32),
    )(xu, xi, hu, hi, wa, wb, b1, w2t, b2, w3, b3)


def kernel(user_ids, item_ids, user_table, item_table, W1, b1, W2, b2, W3, b3):
    packed = _pack(user_table.T, item_table.T)
    uid = user_ids.astype(jnp.int32)
    iid = item_ids.astype(jnp.int32)
    uq = ((uid >> SH_BLK) << SH_HALF) | (uid & (PACK_HALF - 1))
    iq = ((iid >> SH_BLK) << SH_HALF) | (iid & (PACK_HALF - 1))
    uq2 = uq.reshape(BATCH // IDX_CHUNK, IDX_CHUNK)
    iq2 = iq.reshape(BATCH // IDX_CHUNK, IDX_CHUNK)
    xu, xi = _make_gather()(uq2, iq2, packed)
    hu = ((uid >> SH_HALF) & 1).astype(jnp.uint32).reshape(BATCH, 1)
    hi = ((iid >> SH_HALF) & 1).astype(jnp.uint32).reshape(BATCH, 1)
    zeros = jnp.zeros((EMBED, 128), jnp.float32)
    wa = jnp.concatenate([W1[:, :EMBED].T, zeros], axis=0)
    wb = jnp.concatenate([zeros, W1[:, EMBED:].T], axis=0)
    return _mlp(xu, xi, hu, hi, wa, wb, b1.reshape(1, 128), W2.T,
                b2.reshape(1, EMBED), W3, b3)


# speedup vs baseline: 2.4708x; 1.0239x over previous
import functools

import jax
import jax.numpy as jnp
from jax import lax
from jax.experimental import pallas as pl
from jax.experimental.pallas import tpu as pltpu
from jax.experimental.pallas import tpu_sc as plsc

EMBED = 64
BATCH = 16384
NROWS = 1000000
NC = 2
NS = 16
NW = NC * NS
B_PER_W = BATCH // NW
IDX_CHUNK = 128
N_CHUNKS = B_PER_W // IDX_CHUNK

PACK_L = 32768
PACK_GRID = -(-NROWS // PACK_L)
NPAIR = PACK_GRID * (PACK_L // 2)
PACK_HALF = PACK_L // 2
SH_HALF = PACK_HALF.bit_length() - 1
SH_BLK = SH_HALF + 1


def _pack_body(u_ref, i_ref, out_ref):
    x = jnp.concatenate([u_ref[...], i_ref[...]], axis=0)
    xt = jnp.swapaxes(x, 0, 1)
    w = lax.bitcast_convert_type(xt, jnp.uint32)
    t = w + jnp.uint32(0x7FFF) + ((w >> 16) & jnp.uint32(1))
    hi = t[: PACK_L // 2, :]
    lo = t[PACK_L // 2:, :]
    out_ref[...] = (hi & jnp.uint32(0xFFFF0000)) | (lo >> 16)


def _pack(u_t, i_t):
    return pl.pallas_call(
        _pack_body,
        grid=(PACK_GRID,),
        in_specs=[
            pl.BlockSpec((EMBED, PACK_L), lambda k: (0, k)),
            pl.BlockSpec((EMBED, PACK_L), lambda k: (0, k)),
        ],
        out_specs=pl.BlockSpec((PACK_L // 2, 2 * EMBED), lambda k: (k, 0)),
        out_shape=jax.ShapeDtypeStruct((NPAIR, 2 * EMBED), jnp.uint32),
    )(u_t, i_t)


def _gather_body(uid_hbm, iid_hbm, packed_hbm, xu_hbm, xi_hbm,
                 uidx_v, iidx_v, rows_v, sem):
    wid = lax.axis_index("s") * NC + lax.axis_index("c")
    base = wid * B_PER_W
    row_base = wid * N_CHUNKS
    pltpu.sync_copy(uid_hbm.at[pl.ds(row_base, N_CHUNKS)], uidx_v)
    pltpu.sync_copy(iid_hbm.at[pl.ds(row_base, N_CHUNKS)], iidx_v)
    for idx_v, dst in ((uidx_v, xu_hbm), (iidx_v, xi_hbm)):
        copies = []
        for j in range(N_CHUNKS):
            copies.append(pltpu.async_copy(
                packed_hbm.at[idx_v.at[j]],
                rows_v.at[pl.ds(j * IDX_CHUNK, IDX_CHUNK)], sem))
        for c in copies:
            c.wait()
        pltpu.sync_copy(rows_v, dst.at[pl.ds(base, B_PER_W)])


@functools.cache
def _make_gather():
    return pl.kernel(
        _gather_body,
        mesh=plsc.VectorSubcoreMesh(core_axis_name="c", subcore_axis_name="s"),
        out_type=[
            jax.ShapeDtypeStruct((BATCH, 2 * EMBED), jnp.uint32),
            jax.ShapeDtypeStruct((BATCH, 2 * EMBED), jnp.uint32),
        ],
        scratch_types=[
            pltpu.VMEM((N_CHUNKS, IDX_CHUNK), jnp.int32),
            pltpu.VMEM((N_CHUNKS, IDX_CHUNK), jnp.int32),
            pltpu.VMEM((B_PER_W, 2 * EMBED), jnp.uint32),
            pltpu.SemaphoreType.DMA,
        ],
    )


def _select_rows(xraw, parity):
    sel = jnp.where(parity == 1, xraw << 16, xraw)
    sel = sel & jnp.uint32(0xFFFF0000)
    return lax.bitcast_convert_type(sel, jnp.float32)


def _mlp_body(xu_ref, xi_ref, hu_ref, hi_ref, wa_ref, wb_ref, b1_ref,
              w2_ref, b2_ref, w3_ref, b3_ref, out_ref):
    xu = _select_rows(xu_ref[...], hu_ref[...])
    xi = _select_rows(xi_ref[...], hi_ref[...])
    h = jnp.dot(xu, wa_ref[...], preferred_element_type=jnp.float32)
    h = h + jnp.dot(xi, wb_ref[...], preferred_element_type=jnp.float32)
    h = jnp.maximum(h + b1_ref[...], 0.0)
    h = jnp.maximum(
        jnp.dot(h, w2_ref[...], preferred_element_type=jnp.float32)
        + b2_ref[...], 0.0)
    out_ref[...] = jnp.sum(h * w3_ref[...], axis=1) + b3_ref[0]


MLP_BLK = 2048


def _mlp(xu, xi, hu, hi, wa, wb, b1, w2t, b2, w3, b3):
    grid = (BATCH // MLP_BLK,)
    return pl.pallas_call(
        _mlp_body,
        grid=grid,
        in_specs=[
            pl.BlockSpec((MLP_BLK, 2 * EMBED), lambda i: (i, 0)),
            pl.BlockSpec((MLP_BLK, 2 * EMBED), lambda i: (i, 0)),
            pl.BlockSpec((MLP_BLK, 1), lambda i: (i, 0)),
            pl.BlockSpec((MLP_BLK, 1), lambda i: (i, 0)),
            pl.BlockSpec((2 * EMBED, 128), lambda i: (0, 0)),
            pl.BlockSpec((2 * EMBED, 128), lambda i: (0, 0)),
            pl.BlockSpec((1, 128), lambda i: (0, 0)),
            pl.BlockSpec((128, EMBED), lambda i: (0, 0)),
            pl.BlockSpec((1, EMBED), lambda i: (0, 0)),
            pl.BlockSpec((1, EMBED), lambda i: (0, 0)),
            pl.BlockSpec(memory_space=pltpu.SMEM),
        ],
        out_specs=pl.BlockSpec((MLP_BLK,), lambda i: (i,)),
        out_shape=jax.ShapeDtypeStruct((BATCH,), jnp.float32),
    )(xu, xi, hu, hi, wa, wb, b1, w2t, b2, w3, b3)


def kernel(user_ids, item_ids, user_table, item_table, W1, b1, W2, b2, W3, b3):
    packed = _pack(user_table.T, item_table.T)
    uid = user_ids.astype(jnp.int32)
    iid = item_ids.astype(jnp.int32)
    uq = ((uid >> SH_BLK) << SH_HALF) | (uid & (PACK_HALF - 1))
    iq = ((iid >> SH_BLK) << SH_HALF) | (iid & (PACK_HALF - 1))
    uq2 = uq.reshape(BATCH // IDX_CHUNK, IDX_CHUNK)
    iq2 = iq.reshape(BATCH // IDX_CHUNK, IDX_CHUNK)
    xu, xi = _make_gather()(uq2, iq2, packed)
    hu = ((uid >> SH_HALF) & 1).astype(jnp.uint32).reshape(BATCH, 1)
    hi = ((iid >> SH_HALF) & 1).astype(jnp.uint32).reshape(BATCH, 1)
    zeros = jnp.zeros((EMBED, 128), jnp.float32)
    wa = jnp.concatenate([W1[:, :EMBED].T, zeros], axis=0)
    wb = jnp.concatenate([zeros, W1[:, EMBED:].T], axis=0)
    return _mlp(xu, xi, hu, hi, wa, wb, b1.reshape(1, 128), W2.T,
                b2.reshape(1, EMBED), W3, b3)
